# baseline (speedup 1.0000x reference)
import math
import functools

import jax
import jax.numpy as jnp
from jax.experimental import pallas as pl
from jax.experimental.pallas import tpu as pltpu


def _gate_kernel(x_ref, wg_ref, comb_ref, p_ref, laux_ref,
                 base_ref, me_ref, ce_ref, *, tb, num_experts, capacity,
                 num_tokens):
    i = pl.program_id(0)
    E = num_experts
    C = capacity

    @pl.when(i == 0)
    def _init():
        base_ref[...] = jnp.zeros_like(base_ref)
        me_ref[...] = jnp.zeros_like(me_ref)
        ce_ref[...] = jnp.zeros_like(ce_ref)

    x = x_ref[...]
    wg = wg_ref[...]
    logits = jax.lax.dot_general(
        wg, x, (((1,), (1,)), ((), ())),
        preferred_element_type=jnp.float32)

    m = jnp.max(logits, axis=0, keepdims=True)
    ex = jnp.exp(logits - m)
    gates = ex / jnp.sum(ex, axis=0, keepdims=True)

    gmax = jnp.max(gates, axis=0, keepdims=True)
    iota_e = jax.lax.broadcasted_iota(jnp.int32, (E, tb), 0)
    idx = jnp.min(jnp.where(gates == gmax, iota_e, E),
                  axis=0, keepdims=True)
    mask1 = (iota_e == idx).astype(jnp.float32)

    me_ref[...] += gates
    ce_ref[...] += mask1

    r = jax.lax.broadcasted_iota(jnp.int32, (tb, tb), 0)
    c = jax.lax.broadcasted_iota(jnp.int32, (tb, tb), 1)
    ut = (r <= c).astype(jnp.float32)
    csum = jax.lax.dot_general(
        mask1, ut, (((1,), (0,)), ((), ())),
        preferred_element_type=jnp.float32)

    locations = base_ref[...] + csum - 1.0
    base_ref[...] += csum[:, tb - 1:tb]

    keep = mask1 * (locations < C).astype(jnp.float32)
    loc_s = jnp.sum(locations * keep, axis=0, keepdims=True)
    gate_s = jnp.sum(gates * keep, axis=0, keepdims=True)
    kept = jnp.sum(keep, axis=0, keepdims=True)

    p = jnp.where(kept > 0.0,
                  idx.astype(jnp.float32) * C + loc_s,
                  -1.0).astype(jnp.int32)
    p_ref[...] = p.reshape(1, 1, tb)

    p_col = p.reshape(tb, 1)[:, :, None]
    g_col = gate_s.reshape(tb, 1)[:, :, None]

    iota_e3 = jax.lax.broadcasted_iota(jnp.int32, (tb, E, 1), 1)
    pe = p_col - iota_e3 * C
    iota_c3 = jax.lax.broadcasted_iota(jnp.int32, (tb, E, C), 2)
    msk = iota_c3 == pe
    comb_ref[...] = jnp.where(msk, g_col, 0.0)

    @pl.when(i == pl.num_programs(0) - 1)
    def _laux():
        me = jnp.sum(me_ref[...], axis=1, keepdims=True) / num_tokens
        ce = jnp.sum(ce_ref[...], axis=1, keepdims=True) / num_tokens
        laux_ref[...] = (jnp.sum(me * ce) * E).reshape(1, 1)


@jax.jit
def kernel(input, wg):
    num_tokens, model_dim = input.shape
    num_experts = wg.shape[0]
    capacity = int(math.ceil(num_tokens / num_experts))
    tb = 256
    num_blocks = num_tokens // tb

    body = functools.partial(
        _gate_kernel, tb=tb, num_experts=num_experts, capacity=capacity,
        num_tokens=num_tokens)

    comb, p_out, laux = pl.pallas_call(
        body,
        grid=(num_blocks,),
        in_specs=[
            pl.BlockSpec((tb, model_dim), lambda i: (i, 0)),
            pl.BlockSpec((num_experts, model_dim), lambda i: (0, 0)),
        ],
        out_specs=[
            pl.BlockSpec((tb, num_experts, capacity), lambda i: (i, 0, 0)),
            pl.BlockSpec((1, 1, tb), lambda i: (i, 0, 0)),
            pl.BlockSpec((1, 1), lambda i: (0, 0)),
        ],
        out_shape=[
            jax.ShapeDtypeStruct((num_tokens, num_experts, capacity),
                                 jnp.float32),
            jax.ShapeDtypeStruct((num_blocks, 1, tb), jnp.int32),
            jax.ShapeDtypeStruct((1, 1), jnp.float32),
        ],
        scratch_shapes=[
            pltpu.VMEM((num_experts, 1), jnp.float32),
            pltpu.VMEM((num_experts, tb), jnp.float32),
            pltpu.VMEM((num_experts, tb), jnp.float32),
        ],
    )(input, wg)

    p_flat = p_out.reshape(num_tokens)
    e_oh = (p_flat[:, None] // capacity) == jnp.arange(num_experts,
                                                      dtype=jnp.int32)[None, :]
    c_oh = (p_flat[:, None] % capacity) == jnp.arange(capacity,
                                                      dtype=jnp.int32)[None, :]
    disp = e_oh[:, :, None] & c_oh[:, None, :]
    return (laux.reshape(()), comb, disp)


# speedup vs baseline: 1.5223x; 1.5223x over previous
import math
import functools

import jax
import jax.numpy as jnp
from jax.experimental import pallas as pl
from jax.experimental.pallas import tpu as pltpu


def _gate_kernel(x_ref, wg_ref, comb_ref, p_ref, laux_ref,
                 base_ref, me_ref, ce_ref, *, tb, num_experts, capacity,
                 num_tokens):
    i = pl.program_id(0)
    E = num_experts
    C = capacity

    @pl.when(i == 0)
    def _init():
        base_ref[...] = jnp.zeros_like(base_ref)
        me_ref[...] = jnp.zeros_like(me_ref)
        ce_ref[...] = jnp.zeros_like(ce_ref)

    x = x_ref[...]
    wg = wg_ref[...]
    logits = jax.lax.dot_general(
        wg, x, (((1,), (1,)), ((), ())),
        preferred_element_type=jnp.float32)

    m = jnp.max(logits, axis=0, keepdims=True)
    ex = jnp.exp(logits - m)
    gates = ex / jnp.sum(ex, axis=0, keepdims=True)

    gmax = jnp.max(gates, axis=0, keepdims=True)
    iota_e = jax.lax.broadcasted_iota(jnp.int32, (E, tb), 0)
    idx = jnp.min(jnp.where(gates == gmax, iota_e, E),
                  axis=0, keepdims=True)
    mask1 = (iota_e == idx).astype(jnp.float32)

    me_ref[...] += gates
    ce_ref[...] += mask1

    r = jax.lax.broadcasted_iota(jnp.int32, (tb, tb), 0)
    c = jax.lax.broadcasted_iota(jnp.int32, (tb, tb), 1)
    ut = (r <= c).astype(jnp.float32)
    csum = jax.lax.dot_general(
        mask1, ut, (((1,), (0,)), ((), ())),
        preferred_element_type=jnp.float32)

    locations = base_ref[...] + csum - 1.0
    base_ref[...] += csum[:, tb - 1:tb]

    keep = mask1 * (locations < C).astype(jnp.float32)
    loc_s = jnp.sum(locations * keep, axis=0, keepdims=True)
    gate_s = jnp.sum(gates * keep, axis=0, keepdims=True)
    kept = jnp.sum(keep, axis=0, keepdims=True)

    p = jnp.where(kept > 0.0,
                  idx.astype(jnp.float32) * C + loc_s,
                  -1.0).astype(jnp.int32)
    p_ref[...] = p.reshape(1, 1, tb)

    p_col = p.reshape(tb, 1)[:, :, None]
    g_col = gate_s.reshape(tb, 1)[:, :, None]

    iota_e3 = jax.lax.broadcasted_iota(jnp.int32, (tb, E, 1), 1)
    pe = p_col - iota_e3 * C
    iota_c3 = jax.lax.broadcasted_iota(jnp.int32, (tb, E, C), 2)
    msk = iota_c3 == pe
    comb_ref[...] = jnp.where(msk, g_col, 0.0)

    @pl.when(i == pl.num_programs(0) - 1)
    def _laux():
        me = jnp.sum(me_ref[...], axis=1, keepdims=True) / num_tokens
        ce = jnp.sum(ce_ref[...], axis=1, keepdims=True) / num_tokens
        laux_ref[...] = (jnp.sum(me * ce) * E).reshape(1, 1)


@jax.jit
def kernel(input, wg):
    num_tokens, model_dim = input.shape
    num_experts = wg.shape[0]
    capacity = int(math.ceil(num_tokens / num_experts))
    tb = 512
    num_blocks = num_tokens // tb

    body = functools.partial(
        _gate_kernel, tb=tb, num_experts=num_experts, capacity=capacity,
        num_tokens=num_tokens)

    comb, p_out, laux = pl.pallas_call(
        body,
        grid=(num_blocks,),
        in_specs=[
            pl.BlockSpec((tb, model_dim), lambda i: (i, 0)),
            pl.BlockSpec((num_experts, model_dim), lambda i: (0, 0)),
        ],
        out_specs=[
            pl.BlockSpec((tb, num_experts, capacity), lambda i: (i, 0, 0)),
            pl.BlockSpec((1, 1, tb), lambda i: (i, 0, 0)),
            pl.BlockSpec((1, 1), lambda i: (0, 0)),
        ],
        out_shape=[
            jax.ShapeDtypeStruct((num_tokens, num_experts, capacity),
                                 jnp.float32),
            jax.ShapeDtypeStruct((num_blocks, 1, tb), jnp.int32),
            jax.ShapeDtypeStruct((1, 1), jnp.float32),
        ],
        scratch_shapes=[
            pltpu.VMEM((num_experts, 1), jnp.float32),
            pltpu.VMEM((num_experts, tb), jnp.float32),
            pltpu.VMEM((num_experts, tb), jnp.float32),
        ],
    )(input, wg)

    p_flat = p_out.reshape(num_tokens)
    ec = (jnp.arange(num_experts, dtype=jnp.int32)[:, None] * capacity
          + jnp.arange(capacity, dtype=jnp.int32)[None, :])
    disp = p_flat[:, None, None] == ec[None]
    return (laux.reshape(()), comb, disp)


# speedup vs baseline: 1.5500x; 1.0182x over previous
import math
import functools

import jax
import jax.numpy as jnp
from jax.experimental import pallas as pl
from jax.experimental.pallas import tpu as pltpu


def _gate_kernel(x_ref, wg_ref, comb_ref, p_ref, laux_ref,
                 base_ref, me_ref, ce_ref, *, tb, num_experts, capacity,
                 num_tokens):
    i = pl.program_id(0)
    E = num_experts
    C = capacity

    @pl.when(i == 0)
    def _init():
        base_ref[...] = jnp.zeros_like(base_ref)
        me_ref[...] = jnp.zeros_like(me_ref)
        ce_ref[...] = jnp.zeros_like(ce_ref)

    x = x_ref[...]
    wg = wg_ref[...]
    logits = jax.lax.dot_general(
        wg, x, (((1,), (1,)), ((), ())),
        preferred_element_type=jnp.float32)

    m = jnp.max(logits, axis=0, keepdims=True)
    ex = jnp.exp(logits - m)
    gates = ex / jnp.sum(ex, axis=0, keepdims=True)

    gmax = jnp.max(gates, axis=0, keepdims=True)
    iota_e = jax.lax.broadcasted_iota(jnp.int32, (E, tb), 0)
    idx = jnp.min(jnp.where(gates == gmax, iota_e, E),
                  axis=0, keepdims=True)
    mask1 = (iota_e == idx).astype(jnp.float32)

    me_ref[...] += gates
    ce_ref[...] += mask1

    r = jax.lax.broadcasted_iota(jnp.int32, (tb, tb), 0)
    c = jax.lax.broadcasted_iota(jnp.int32, (tb, tb), 1)
    ut = (r <= c).astype(jnp.float32)
    csum = jax.lax.dot_general(
        mask1, ut, (((1,), (0,)), ((), ())),
        preferred_element_type=jnp.float32)

    locations = base_ref[...] + csum - 1.0
    base_ref[...] += csum[:, tb - 1:tb]

    keep = mask1 * (locations < C).astype(jnp.float32)
    loc_s = jnp.sum(locations * keep, axis=0, keepdims=True)
    gate_s = jnp.sum(gates * keep, axis=0, keepdims=True)
    kept = jnp.sum(keep, axis=0, keepdims=True)

    p = jnp.where(kept > 0.0,
                  idx.astype(jnp.float32) * C + loc_s,
                  -1.0).astype(jnp.int32)
    p_ref[...] = p.reshape(1, 1, tb)

    p_col = p.reshape(tb, 1)[:, :, None]
    g_col = gate_s.reshape(tb, 1)[:, :, None]

    iota_e3 = jax.lax.broadcasted_iota(jnp.int32, (tb, E, 1), 1)
    pe = p_col - iota_e3 * C
    iota_c3 = jax.lax.broadcasted_iota(jnp.int32, (tb, E, C), 2)
    msk = iota_c3 == pe
    comb_ref[...] = jnp.where(msk, g_col, 0.0)

    @pl.when(i == pl.num_programs(0) - 1)
    def _laux():
        me = jnp.sum(me_ref[...], axis=1, keepdims=True) / num_tokens
        ce = jnp.sum(ce_ref[...], axis=1, keepdims=True) / num_tokens
        laux_ref[...] = (jnp.sum(me * ce) * E).reshape(1, 1)


@jax.jit
def kernel(input, wg):
    num_tokens, model_dim = input.shape
    num_experts = wg.shape[0]
    capacity = int(math.ceil(num_tokens / num_experts))
    tb = 1024
    num_blocks = num_tokens // tb

    body = functools.partial(
        _gate_kernel, tb=tb, num_experts=num_experts, capacity=capacity,
        num_tokens=num_tokens)

    comb, p_out, laux = pl.pallas_call(
        body,
        grid=(num_blocks,),
        in_specs=[
            pl.BlockSpec((tb, model_dim), lambda i: (i, 0)),
            pl.BlockSpec((num_experts, model_dim), lambda i: (0, 0)),
        ],
        out_specs=[
            pl.BlockSpec((tb, num_experts, capacity), lambda i: (i, 0, 0)),
            pl.BlockSpec((1, 1, tb), lambda i: (i, 0, 0)),
            pl.BlockSpec((1, 1), lambda i: (0, 0)),
        ],
        out_shape=[
            jax.ShapeDtypeStruct((num_tokens, num_experts, capacity),
                                 jnp.float32),
            jax.ShapeDtypeStruct((num_blocks, 1, tb), jnp.int32),
            jax.ShapeDtypeStruct((1, 1), jnp.float32),
        ],
        scratch_shapes=[
            pltpu.VMEM((num_experts, 1), jnp.float32),
            pltpu.VMEM((num_experts, tb), jnp.float32),
            pltpu.VMEM((num_experts, tb), jnp.float32),
        ],
    )(input, wg)

    p_flat = p_out.reshape(num_tokens)
    ec = (jnp.arange(num_experts, dtype=jnp.int32)[:, None] * capacity
          + jnp.arange(capacity, dtype=jnp.int32)[None, :])
    disp = p_flat[:, None, None] == ec[None]
    return (laux.reshape(()), comb, disp)
